# Initial kernel scaffold; baseline (speedup 1.0000x reference)
import jax
import jax.numpy as jnp
from jax.experimental import pallas as pl


def kernel(token_ids, weight):
    raise NotImplementedError("write your pallas kernel here")


# speedup vs baseline: 1.6829x; 1.6829x over previous
import functools

import jax
import jax.numpy as jnp
from jax import lax
from jax.experimental import pallas as pl
from jax.experimental.pallas import tpu as pltpu
from jax.experimental.pallas import tpu_sc as plsc

_D = 64
_CHUNK = 128

_INFO = plsc.get_sparse_core_info()
_NC = _INFO.num_cores
_NS = _INFO.num_subcores
_NW = _NC * _NS


def _emb_body(n_chunks, idx_hbm, table_hbm, out_hbm, idx_v, rows_v, gsem):
    wid = lax.axis_index("s") * _NC + lax.axis_index("c")
    pltpu.sync_copy(idx_hbm.at[wid], idx_v)
    row_base = wid * n_chunks * _CHUNK

    def step(j, _):
        pltpu.async_copy(table_hbm.at[idx_v.at[j]], rows_v, gsem).wait()
        pltpu.sync_copy(rows_v, out_hbm.at[pl.ds(row_base + j * _CHUNK, _CHUNK)])
        return 0

    lax.fori_loop(0, n_chunks, step, 0)


@functools.partial(jax.jit, static_argnames=("n_chunks",))
def _emb_call(idx, weight, n_chunks):
    total = _NW * n_chunks * _CHUNK
    mesh = plsc.VectorSubcoreMesh(core_axis_name="c", subcore_axis_name="s")
    run = pl.kernel(
        functools.partial(_emb_body, n_chunks),
        out_type=jax.ShapeDtypeStruct((total, _D), jnp.float32),
        mesh=mesh,
        scratch_types=[
            pltpu.VMEM((n_chunks, _CHUNK), jnp.int32),
            pltpu.VMEM((_CHUNK, _D), jnp.float32),
            pltpu.SemaphoreType.DMA,
        ],
        compiler_params=pltpu.CompilerParams(use_tc_tiling_on_sc=False),
    )
    return run(idx, weight)


def kernel(token_ids, weight):
    b, h = token_ids.shape
    total = b * h
    flat = token_ids.reshape(total).astype(jnp.int32)
    per_w = total // _NW
    n_chunks = per_w // _CHUNK
    idx = flat.reshape(_NW, n_chunks, _CHUNK)
    out = _emb_call(idx, weight, n_chunks)
    return out.reshape(b, h, _D)


# speedup vs baseline: 1.8743x; 1.1137x over previous
import functools

import jax
import jax.numpy as jnp
from jax import lax
from jax.experimental import pallas as pl
from jax.experimental.pallas import tpu as pltpu
from jax.experimental.pallas import tpu_sc as plsc

_D = 64
_CHUNK = 128
_NBUF = 8
_LOOKAHEAD = 5

_INFO = plsc.get_sparse_core_info()
_NC = _INFO.num_cores
_NS = _INFO.num_subcores
_NW = _NC * _NS


def _emb_body(n_chunks, idx_hbm, table_hbm, out_hbm, idx_v, rows_v, gsem, osem):
    wid = lax.axis_index("s") * _NC + lax.axis_index("c")
    pltpu.sync_copy(idx_hbm.at[wid], idx_v)
    row_base = wid * n_chunks * _CHUNK

    def issue_gather(k, b):
        pltpu.async_copy(table_hbm.at[idx_v.at[k]], rows_v.at[b], gsem.at[b])

    def wait_gather(k, b):
        pltpu.make_async_copy(
            table_hbm.at[idx_v.at[k]], rows_v.at[b], gsem.at[b]).wait()

    def out_slice(j):
        return out_hbm.at[pl.ds(row_base + j * _CHUNK, _CHUNK)]

    def issue_out(j, b):
        pltpu.async_copy(rows_v.at[b], out_slice(j), osem.at[b])

    def wait_out(j, b):
        pltpu.make_async_copy(rows_v.at[b], out_slice(j), osem.at[b]).wait()

    for b in range(_LOOKAHEAD):
        issue_gather(b, b)

    def group(g, carry):
        for s in range(_NBUF):
            j = g * _NBUF + s
            wait_gather(j, s)
            issue_out(j, s)

            @pl.when(j >= 2)
            def _():
                wait_out(j - 2, (s - 2) % _NBUF)

            k = j + _LOOKAHEAD

            @pl.when(k < n_chunks)
            def _():
                issue_gather(k, (s + _LOOKAHEAD) % _NBUF)
        return carry

    lax.fori_loop(0, n_chunks // _NBUF, group, 0)

    wait_out(n_chunks - 2, (n_chunks - 2) % _NBUF)
    wait_out(n_chunks - 1, (n_chunks - 1) % _NBUF)


@functools.partial(jax.jit, static_argnames=("n_chunks",))
def _emb_call(idx, weight, n_chunks):
    total = _NW * n_chunks * _CHUNK
    mesh = plsc.VectorSubcoreMesh(core_axis_name="c", subcore_axis_name="s")
    run = pl.kernel(
        functools.partial(_emb_body, n_chunks),
        out_type=jax.ShapeDtypeStruct((total, _D), jnp.float32),
        mesh=mesh,
        scratch_types=[
            pltpu.VMEM((n_chunks, _CHUNK), jnp.int32),
            pltpu.VMEM((_NBUF, _CHUNK, _D), jnp.float32),
            pltpu.SemaphoreType.DMA((_NBUF,)),
            pltpu.SemaphoreType.DMA((_NBUF,)),
        ],
        compiler_params=pltpu.CompilerParams(use_tc_tiling_on_sc=False),
    )
    return run(idx, weight)


def kernel(token_ids, weight):
    b, h = token_ids.shape
    total = b * h
    flat = token_ids.reshape(total).astype(jnp.int32)
    per_w = total // _NW
    n_chunks = per_w // _CHUNK
    idx = flat.reshape(_NW, n_chunks, _CHUNK)
    out = _emb_call(idx, weight, n_chunks)
    return out.reshape(b, h, _D)
